# baseline (speedup 1.0000x reference)
import functools

import jax
import jax.numpy as jnp
from jax import lax
from jax.experimental import pallas as pl
from jax.experimental.pallas import tpu as pltpu
from jax.experimental.pallas import tpu_sc as plsc

N_NODES = 10000
N_EDGES = 320000
CH = 128
HID = 32

NC = 2
NS = 16
NW = NC * NS
EPW = N_EDGES // NW
K = 80
NCHUNK = EPW // K
NPAD = 10240
K_D = 125
NCHUNK_D = EPW // K_D
ZR = 80
RPT = NPAD // NS
DEGW = 128

_sc_mesh = plsc.VectorSubcoreMesh(core_axis_name="c", subcore_axis_name="s")


@functools.partial(
    pl.kernel,
    out_type=jax.ShapeDtypeStruct((NC, NPAD, DEGW), jnp.float32),
    mesh=_sc_mesh,
    scratch_types=[
        pltpu.VMEM((NCHUNK_D, K_D), jnp.int32),
        pltpu.VMEM((K_D, DEGW), jnp.float32),
        pltpu.VMEM_SHARED((NPAD, DEGW), jnp.float32),
    ],
)
def _sc_deg(dst_hbm, ones_hbm, zeros_hbm, out_hbm, idx_v, ones_v, deg_sh):
    c = lax.axis_index("c")
    s = lax.axis_index("s")
    wid = c * NS + s
    pltpu.sync_copy(dst_hbm.at[wid], idx_v)
    for t in range(RPT // ZR):
        pltpu.sync_copy(zeros_hbm, deg_sh.at[pl.ds(s * RPT + t * ZR, ZR)])
    pltpu.sync_copy(ones_hbm, ones_v)
    plsc.subcore_barrier()

    def body(j, carry):
        pltpu.sync_copy(ones_v, deg_sh.at[idx_v.at[j]], add=True)
        return carry

    lax.fori_loop(0, NCHUNK_D, body, 0)
    plsc.subcore_barrier()
    pltpu.sync_copy(deg_sh.at[pl.ds(s * RPT, RPT)],
                    out_hbm.at[c, pl.ds(s * RPT, RPT)])


@functools.partial(
    pl.kernel,
    out_type=jax.ShapeDtypeStruct((NC, NPAD, CH), jnp.float32),
    mesh=_sc_mesh,
    scratch_types=[
        pltpu.VMEM((EPW,), jnp.int32),
        pltpu.VMEM((3, K), jnp.int32),
        pltpu.VMEM((3, K, CH), jnp.float32),
        pltpu.VMEM_SHARED((NPAD, CH), jnp.float32),
        pltpu.SemaphoreType.DMA,
        pltpu.SemaphoreType.DMA,
        pltpu.SemaphoreType.DMA,
        pltpu.SemaphoreType.DMA,
        pltpu.SemaphoreType.DMA,
        pltpu.SemaphoreType.DMA,
    ],
)
def _sc_scatter(xws_hbm, src_hbm, dst_hbm, zeros_hbm, out_hbm,
                idxs_v, idxd_v, rows_v, acc_sh,
                gsem0, gsem1, gsem2, dsem0, dsem1, dsem2):
    c = lax.axis_index("c")
    s = lax.axis_index("s")
    wid = c * NS + s
    base0 = wid * EPW
    gsems = (gsem0, gsem1, gsem2)
    dsems = (dsem0, dsem1, dsem2)
    pltpu.sync_copy(src_hbm.at[pl.ds(base0, EPW)], idxs_v)
    for t in range(RPT // ZR):
        pltpu.sync_copy(zeros_hbm, acc_sh.at[pl.ds(s * RPT + t * ZR, ZR)])
    plsc.subcore_barrier()

    def issue(j, b):
        pltpu.async_copy(dst_hbm.at[pl.ds(base0 + j * K, K)], idxd_v.at[b],
                         dsems[b])
        pltpu.async_copy(xws_hbm.at[idxs_v.at[pl.ds(j * K, K)]],
                         rows_v.at[b], gsems[b])

    def drain_scatter(j, b):
        pltpu.make_async_copy(dst_hbm.at[pl.ds(base0 + j * K, K)],
                              idxd_v.at[b], dsems[b]).wait()
        pltpu.make_async_copy(xws_hbm.at[idxs_v.at[pl.ds(j * K, K)]],
                              rows_v.at[b], gsems[b]).wait()
        pltpu.sync_copy(rows_v.at[b], acc_sh.at[idxd_v.at[b]], add=True)

    for b in range(3):
        issue(b, b)

    NSTEADY = (NCHUNK - 2) // 3

    def body(i, carry):
        for b in range(3):
            j = 3 * i + b
            drain_scatter(j, b)
            issue(j + 3, b)
        return carry

    lax.fori_loop(0, NSTEADY - 1, body, 0)
    for jj in range(3 * (NSTEADY - 1), NCHUNK):
        drain_scatter(jj, jj % 3)
        if jj + 3 < NCHUNK:
            issue(jj + 3, jj % 3)
    plsc.subcore_barrier()
    pltpu.sync_copy(acc_sh.at[pl.ds(s * RPT, RPT)],
                    out_hbm.at[c, pl.ds(s * RPT, RPT)])


_R = 2000


def _prep_body(state_ref, w_ref, degp_ref, xws_ref):
    xw = jnp.dot(state_ref[...], w_ref[...], preferred_element_type=jnp.float32)
    degp = degp_ref[...]
    deg = degp[0, :, 0] + degp[1, :, 0] + 1.0
    dinv = lax.rsqrt(deg)
    xws_ref[...] = xw * dinv[:, None]


def _head_body(accp_ref, xws_ref, state_ref, degp_ref, bg_ref,
               w1_ref, b1_ref, w2_ref, b2_ref, w3_ref, b3_ref, out_ref):
    accp = accp_ref[...]
    xws = xws_ref[...]
    degp = degp_ref[...]
    deg = degp[0, :, 0] + degp[1, :, 0] + 1.0
    dinv = lax.rsqrt(deg)[:, None]
    total = dinv * (accp[0] + accp[1] + xws) + bg_ref[...][None, :]
    h = jnp.maximum(total, 0.0) + state_ref[...]
    t = jnp.maximum(jnp.dot(h, w1_ref[...], preferred_element_type=jnp.float32)
                    + b1_ref[...][None, :], 0.0)
    t = jnp.maximum(jnp.dot(t, w2_ref[...], preferred_element_type=jnp.float32)
                    + b2_ref[...][None, :], 0.0)
    z = (jnp.dot(t, w3_ref[...], preferred_element_type=jnp.float32)
         + b3_ref[...][None, :])
    out_ref[...] = (jnp.maximum(z, 0.0) + jnp.log1p(jnp.exp(-jnp.abs(z)))
                    + 1e-20)


_tc_prep = pl.pallas_call(
    _prep_body,
    grid=(N_NODES // _R,),
    in_specs=[
        pl.BlockSpec((_R, CH), lambda i: (i, 0)),
        pl.BlockSpec((CH, CH), lambda i: (0, 0)),
        pl.BlockSpec((NC, _R, DEGW), lambda i: (0, i, 0)),
    ],
    out_specs=pl.BlockSpec((_R, CH), lambda i: (i, 0)),
    out_shape=jax.ShapeDtypeStruct((N_NODES, CH), jnp.float32),
)

_tc_head = pl.pallas_call(
    _head_body,
    grid=(N_NODES // _R,),
    in_specs=[
        pl.BlockSpec((NC, _R, CH), lambda i: (0, i, 0)),
        pl.BlockSpec((_R, CH), lambda i: (i, 0)),
        pl.BlockSpec((_R, CH), lambda i: (i, 0)),
        pl.BlockSpec((NC, _R, DEGW), lambda i: (0, i, 0)),
        pl.BlockSpec((CH,), lambda i: (0,)),
        pl.BlockSpec((CH, HID), lambda i: (0, 0)),
        pl.BlockSpec((HID,), lambda i: (0,)),
        pl.BlockSpec((HID, HID), lambda i: (0, 0)),
        pl.BlockSpec((HID,), lambda i: (0,)),
        pl.BlockSpec((HID, 1), lambda i: (0, 0)),
        pl.BlockSpec((1,), lambda i: (0,)),
    ],
    out_specs=pl.BlockSpec((_R, 1), lambda i: (i, 0)),
    out_shape=jax.ShapeDtypeStruct((N_NODES, 1), jnp.float32),
)


def kernel(state, edge_index, W_gcn, b_gcn, W1, b1, W2, b2, W3, b3):
    ei = edge_index.astype(jnp.int32)
    dst3 = ei[1].reshape(NW, NCHUNK_D, K_D)
    ones_deg = jnp.ones((K_D, DEGW), jnp.float32)
    zeros_blk = jnp.zeros((ZR, CH), jnp.float32)

    degp = _sc_deg(dst3, ones_deg, zeros_blk)
    xws = _tc_prep(state, W_gcn, degp)
    accp = _sc_scatter(xws, ei[0], ei[1], zeros_blk)
    y = _tc_head(accp, xws, state, degp, b_gcn, W1, b1, W2, b2, W3, b3)
    return y.reshape(N_NODES // 8, 8)


# speedup vs baseline: 1.0017x; 1.0017x over previous
import functools

import jax
import jax.numpy as jnp
from jax import lax
from jax.experimental import pallas as pl
from jax.experimental.pallas import tpu as pltpu
from jax.experimental.pallas import tpu_sc as plsc

N_NODES = 10000
N_EDGES = 320000
CH = 128
HID = 32

NC = 2
NS = 16
NW = NC * NS
EPW = N_EDGES // NW
K = 80
NCHUNK = EPW // K
NPAD = 10240
K_D = 80
NCHUNK_D = EPW // K_D
ZR = 80
RPT = NPAD // NS
DEGW = 128

_sc_mesh = plsc.VectorSubcoreMesh(core_axis_name="c", subcore_axis_name="s")


@functools.partial(
    pl.kernel,
    out_type=jax.ShapeDtypeStruct((NC, NPAD, DEGW), jnp.float32),
    mesh=_sc_mesh,
    scratch_types=[
        pltpu.VMEM((NCHUNK_D, K_D), jnp.int32),
        pltpu.VMEM((K_D, DEGW), jnp.float32),
        pltpu.VMEM_SHARED((NPAD, DEGW), jnp.float32),
    ],
)
def _sc_deg(dst_hbm, ones_hbm, zeros_hbm, out_hbm, idx_v, ones_v, deg_sh):
    c = lax.axis_index("c")
    s = lax.axis_index("s")
    wid = c * NS + s
    pltpu.sync_copy(dst_hbm.at[wid], idx_v)
    for t in range(RPT // ZR):
        pltpu.sync_copy(zeros_hbm, deg_sh.at[pl.ds(s * RPT + t * ZR, ZR)])
    pltpu.sync_copy(ones_hbm, ones_v)
    plsc.subcore_barrier()

    def body(j, carry):
        pltpu.sync_copy(ones_v, deg_sh.at[idx_v.at[j]], add=True)
        return carry

    lax.fori_loop(0, NCHUNK_D, body, 0)
    plsc.subcore_barrier()
    pltpu.sync_copy(deg_sh.at[pl.ds(s * RPT, RPT)],
                    out_hbm.at[c, pl.ds(s * RPT, RPT)])


@functools.partial(
    pl.kernel,
    out_type=jax.ShapeDtypeStruct((NC, NPAD, CH), jnp.float32),
    mesh=_sc_mesh,
    scratch_types=[
        pltpu.VMEM((EPW,), jnp.int32),
        pltpu.VMEM((3, K), jnp.int32),
        pltpu.VMEM((3, K, CH), jnp.float32),
        pltpu.VMEM_SHARED((NPAD, CH), jnp.float32),
        pltpu.SemaphoreType.DMA,
        pltpu.SemaphoreType.DMA,
        pltpu.SemaphoreType.DMA,
        pltpu.SemaphoreType.DMA,
        pltpu.SemaphoreType.DMA,
        pltpu.SemaphoreType.DMA,
    ],
)
def _sc_scatter(xws_hbm, src_hbm, dst_hbm, zeros_hbm, out_hbm,
                idxs_v, idxd_v, rows_v, acc_sh,
                gsem0, gsem1, gsem2, dsem0, dsem1, dsem2):
    c = lax.axis_index("c")
    s = lax.axis_index("s")
    wid = c * NS + s
    base0 = wid * EPW
    gsems = (gsem0, gsem1, gsem2)
    dsems = (dsem0, dsem1, dsem2)
    pltpu.sync_copy(src_hbm.at[pl.ds(base0, EPW)], idxs_v)
    for t in range(RPT // ZR):
        pltpu.sync_copy(zeros_hbm, acc_sh.at[pl.ds(s * RPT + t * ZR, ZR)])
    plsc.subcore_barrier()

    def issue(j, b):
        pltpu.async_copy(dst_hbm.at[pl.ds(base0 + j * K, K)], idxd_v.at[b],
                         dsems[b])
        pltpu.async_copy(xws_hbm.at[idxs_v.at[pl.ds(j * K, K)]],
                         rows_v.at[b], gsems[b])

    def drain_scatter(j, b):
        pltpu.make_async_copy(dst_hbm.at[pl.ds(base0 + j * K, K)],
                              idxd_v.at[b], dsems[b]).wait()
        pltpu.make_async_copy(xws_hbm.at[idxs_v.at[pl.ds(j * K, K)]],
                              rows_v.at[b], gsems[b]).wait()
        pltpu.sync_copy(rows_v.at[b], acc_sh.at[idxd_v.at[b]], add=True)

    for b in range(3):
        issue(b, b)

    NSTEADY = (NCHUNK - 2) // 3

    def body(i, carry):
        for b in range(3):
            j = 3 * i + b
            drain_scatter(j, b)
            issue(j + 3, b)
        return carry

    lax.fori_loop(0, NSTEADY - 1, body, 0)
    for jj in range(3 * (NSTEADY - 1), NCHUNK):
        drain_scatter(jj, jj % 3)
        if jj + 3 < NCHUNK:
            issue(jj + 3, jj % 3)
    plsc.subcore_barrier()
    pltpu.sync_copy(acc_sh.at[pl.ds(s * RPT, RPT)],
                    out_hbm.at[c, pl.ds(s * RPT, RPT)])


_R = 2000


def _prep_body(state_ref, w_ref, degp_ref, xws_ref):
    xw = jnp.dot(state_ref[...], w_ref[...], preferred_element_type=jnp.float32)
    degp = degp_ref[...]
    deg = degp[0, :, 0] + degp[1, :, 0] + 1.0
    dinv = lax.rsqrt(deg)
    xws_ref[...] = xw * dinv[:, None]


def _head_body(accp_ref, xws_ref, state_ref, degp_ref, bg_ref,
               w1_ref, b1_ref, w2_ref, b2_ref, w3_ref, b3_ref, out_ref):
    accp = accp_ref[...]
    xws = xws_ref[...]
    degp = degp_ref[...]
    deg = degp[0, :, 0] + degp[1, :, 0] + 1.0
    dinv = lax.rsqrt(deg)[:, None]
    total = dinv * (accp[0] + accp[1] + xws) + bg_ref[...][None, :]
    h = jnp.maximum(total, 0.0) + state_ref[...]
    t = jnp.maximum(jnp.dot(h, w1_ref[...], preferred_element_type=jnp.float32)
                    + b1_ref[...][None, :], 0.0)
    t = jnp.maximum(jnp.dot(t, w2_ref[...], preferred_element_type=jnp.float32)
                    + b2_ref[...][None, :], 0.0)
    z = (jnp.dot(t, w3_ref[...], preferred_element_type=jnp.float32)
         + b3_ref[...][None, :])
    out_ref[...] = (jnp.maximum(z, 0.0) + jnp.log1p(jnp.exp(-jnp.abs(z)))
                    + 1e-20)


_tc_prep = pl.pallas_call(
    _prep_body,
    grid=(N_NODES // _R,),
    in_specs=[
        pl.BlockSpec((_R, CH), lambda i: (i, 0)),
        pl.BlockSpec((CH, CH), lambda i: (0, 0)),
        pl.BlockSpec((NC, _R, DEGW), lambda i: (0, i, 0)),
    ],
    out_specs=pl.BlockSpec((_R, CH), lambda i: (i, 0)),
    out_shape=jax.ShapeDtypeStruct((N_NODES, CH), jnp.float32),
)

_tc_head = pl.pallas_call(
    _head_body,
    grid=(N_NODES // _R,),
    in_specs=[
        pl.BlockSpec((NC, _R, CH), lambda i: (0, i, 0)),
        pl.BlockSpec((_R, CH), lambda i: (i, 0)),
        pl.BlockSpec((_R, CH), lambda i: (i, 0)),
        pl.BlockSpec((NC, _R, DEGW), lambda i: (0, i, 0)),
        pl.BlockSpec((CH,), lambda i: (0,)),
        pl.BlockSpec((CH, HID), lambda i: (0, 0)),
        pl.BlockSpec((HID,), lambda i: (0,)),
        pl.BlockSpec((HID, HID), lambda i: (0, 0)),
        pl.BlockSpec((HID,), lambda i: (0,)),
        pl.BlockSpec((HID, 1), lambda i: (0, 0)),
        pl.BlockSpec((1,), lambda i: (0,)),
    ],
    out_specs=pl.BlockSpec((_R, 1), lambda i: (i, 0)),
    out_shape=jax.ShapeDtypeStruct((N_NODES, 1), jnp.float32),
)


def kernel(state, edge_index, W_gcn, b_gcn, W1, b1, W2, b2, W3, b3):
    ei = edge_index.astype(jnp.int32)
    dst3 = ei[1].reshape(NW, NCHUNK_D, K_D)
    ones_deg = jnp.ones((K_D, DEGW), jnp.float32)
    zeros_blk = jnp.zeros((ZR, CH), jnp.float32)

    degp = _sc_deg(dst3, ones_deg, zeros_blk)
    xws = _tc_prep(state, W_gcn, degp)
    accp = _sc_scatter(xws, ei[0], ei[1], zeros_blk)
    y = _tc_head(accp, xws, state, degp, b_gcn, W1, b1, W2, b2, W3, b3)
    return y.reshape(N_NODES // 8, 8)


# speedup vs baseline: 1.1061x; 1.1043x over previous
import functools

import jax
import jax.numpy as jnp
from jax import lax
from jax.experimental import pallas as pl
from jax.experimental.pallas import tpu as pltpu
from jax.experimental.pallas import tpu_sc as plsc

N_NODES = 10000
N_EDGES = 320000
CH = 128
HID = 32

NC = 2
NS = 16
NW = NC * NS
EPW = N_EDGES // NW
K = 80
NCHUNK = EPW // K
NPAD = 10240
K_D = 80
NCHUNK_D = EPW // K_D

RPT = NPAD // NS
DEGW = 128

_sc_mesh = plsc.VectorSubcoreMesh(core_axis_name="c", subcore_axis_name="s")


@functools.partial(
    pl.kernel,
    out_type=jax.ShapeDtypeStruct((NC, NPAD, DEGW), jnp.float32),
    mesh=_sc_mesh,
    scratch_types=[
        pltpu.VMEM((NCHUNK_D, K_D), jnp.int32),
        pltpu.VMEM((K_D, DEGW), jnp.float32),
        pltpu.VMEM_SHARED((NPAD, DEGW), jnp.float32),
    ],
)
def _sc_deg(dst_hbm, ones_hbm, zeros_hbm, out_hbm, idx_v, ones_v, deg_sh):
    c = lax.axis_index("c")
    s = lax.axis_index("s")
    wid = c * NS + s
    pltpu.sync_copy(dst_hbm.at[wid], idx_v)
    pltpu.sync_copy(zeros_hbm, deg_sh.at[pl.ds(s * RPT, RPT)])
    pltpu.sync_copy(ones_hbm, ones_v)
    plsc.subcore_barrier()

    def body(j, carry):
        pltpu.sync_copy(ones_v, deg_sh.at[idx_v.at[j]], add=True)
        return carry

    lax.fori_loop(0, NCHUNK_D, body, 0)
    plsc.subcore_barrier()
    pltpu.sync_copy(deg_sh.at[pl.ds(s * RPT, RPT)],
                    out_hbm.at[c, pl.ds(s * RPT, RPT)])


@functools.partial(
    pl.kernel,
    out_type=jax.ShapeDtypeStruct((NC, NPAD, CH), jnp.float32),
    mesh=_sc_mesh,
    scratch_types=[
        pltpu.VMEM((EPW,), jnp.int32),
        pltpu.VMEM((3, K), jnp.int32),
        pltpu.VMEM((3, K, CH), jnp.float32),
        pltpu.VMEM_SHARED((NPAD, CH), jnp.float32),
        pltpu.SemaphoreType.DMA,
        pltpu.SemaphoreType.DMA,
        pltpu.SemaphoreType.DMA,
        pltpu.SemaphoreType.DMA,
        pltpu.SemaphoreType.DMA,
        pltpu.SemaphoreType.DMA,
    ],
)
def _sc_scatter(xws_hbm, src_hbm, dst_hbm, zeros_hbm, out_hbm,
                idxs_v, idxd_v, rows_v, acc_sh,
                gsem0, gsem1, gsem2, dsem0, dsem1, dsem2):
    c = lax.axis_index("c")
    s = lax.axis_index("s")
    wid = c * NS + s
    base0 = wid * EPW
    gsems = (gsem0, gsem1, gsem2)
    dsems = (dsem0, dsem1, dsem2)
    pltpu.sync_copy(src_hbm.at[pl.ds(base0, EPW)], idxs_v)
    pltpu.sync_copy(zeros_hbm, acc_sh.at[pl.ds(s * RPT, RPT)])
    plsc.subcore_barrier()

    def issue(j, b):
        pltpu.async_copy(dst_hbm.at[pl.ds(base0 + j * K, K)], idxd_v.at[b],
                         dsems[b])
        pltpu.async_copy(xws_hbm.at[idxs_v.at[pl.ds(j * K, K)]],
                         rows_v.at[b], gsems[b])

    def drain_scatter(j, b):
        pltpu.make_async_copy(dst_hbm.at[pl.ds(base0 + j * K, K)],
                              idxd_v.at[b], dsems[b]).wait()
        pltpu.make_async_copy(xws_hbm.at[idxs_v.at[pl.ds(j * K, K)]],
                              rows_v.at[b], gsems[b]).wait()
        pltpu.sync_copy(rows_v.at[b], acc_sh.at[idxd_v.at[b]], add=True)

    for b in range(3):
        issue(b, b)

    NSTEADY = (NCHUNK - 2) // 3

    def body(i, carry):
        for b in range(3):
            j = 3 * i + b
            drain_scatter(j, b)
            issue(j + 3, b)
        return carry

    lax.fori_loop(0, NSTEADY - 1, body, 0)
    for jj in range(3 * (NSTEADY - 1), NCHUNK):
        drain_scatter(jj, jj % 3)
        if jj + 3 < NCHUNK:
            issue(jj + 3, jj % 3)
    plsc.subcore_barrier()
    pltpu.sync_copy(acc_sh.at[pl.ds(s * RPT, RPT)],
                    out_hbm.at[c, pl.ds(s * RPT, RPT)])


_R = 2000


def _prep_body(state_ref, w_ref, degp_ref, xws_ref):
    xw = jnp.dot(state_ref[...], w_ref[...], preferred_element_type=jnp.float32)
    degp = degp_ref[...]
    deg = degp[0, :, 0] + degp[1, :, 0] + 1.0
    dinv = lax.rsqrt(deg)
    xws_ref[...] = xw * dinv[:, None]


def _head_body(accp_ref, xws_ref, state_ref, degp_ref, bg_ref,
               w1_ref, b1_ref, w2_ref, b2_ref, w3_ref, b3_ref, out_ref):
    accp = accp_ref[...]
    xws = xws_ref[...]
    degp = degp_ref[...]
    deg = degp[0, :, 0] + degp[1, :, 0] + 1.0
    dinv = lax.rsqrt(deg)[:, None]
    total = dinv * (accp[0] + accp[1] + xws) + bg_ref[...][None, :]
    h = jnp.maximum(total, 0.0) + state_ref[...]
    t = jnp.maximum(jnp.dot(h, w1_ref[...], preferred_element_type=jnp.float32)
                    + b1_ref[...][None, :], 0.0)
    t = jnp.maximum(jnp.dot(t, w2_ref[...], preferred_element_type=jnp.float32)
                    + b2_ref[...][None, :], 0.0)
    z = (jnp.dot(t, w3_ref[...], preferred_element_type=jnp.float32)
         + b3_ref[...][None, :])
    out_ref[...] = (jnp.maximum(z, 0.0) + jnp.log1p(jnp.exp(-jnp.abs(z)))
                    + 1e-20)


_tc_prep = pl.pallas_call(
    _prep_body,
    grid=(N_NODES // _R,),
    in_specs=[
        pl.BlockSpec((_R, CH), lambda i: (i, 0)),
        pl.BlockSpec((CH, CH), lambda i: (0, 0)),
        pl.BlockSpec((NC, _R, DEGW), lambda i: (0, i, 0)),
    ],
    out_specs=pl.BlockSpec((_R, CH), lambda i: (i, 0)),
    out_shape=jax.ShapeDtypeStruct((N_NODES, CH), jnp.float32),
)

_tc_head = pl.pallas_call(
    _head_body,
    grid=(N_NODES // _R,),
    in_specs=[
        pl.BlockSpec((NC, _R, CH), lambda i: (0, i, 0)),
        pl.BlockSpec((_R, CH), lambda i: (i, 0)),
        pl.BlockSpec((_R, CH), lambda i: (i, 0)),
        pl.BlockSpec((NC, _R, DEGW), lambda i: (0, i, 0)),
        pl.BlockSpec((CH,), lambda i: (0,)),
        pl.BlockSpec((CH, HID), lambda i: (0, 0)),
        pl.BlockSpec((HID,), lambda i: (0,)),
        pl.BlockSpec((HID, HID), lambda i: (0, 0)),
        pl.BlockSpec((HID,), lambda i: (0,)),
        pl.BlockSpec((HID, 1), lambda i: (0, 0)),
        pl.BlockSpec((1,), lambda i: (0,)),
    ],
    out_specs=pl.BlockSpec((_R, 1), lambda i: (i, 0)),
    out_shape=jax.ShapeDtypeStruct((N_NODES, 1), jnp.float32),
)


def kernel(state, edge_index, W_gcn, b_gcn, W1, b1, W2, b2, W3, b3):
    ei = edge_index.astype(jnp.int32)
    dst3 = ei[1].reshape(NW, NCHUNK_D, K_D)
    ones_deg = jnp.ones((K_D, DEGW), jnp.float32)
    zeros_blk = jnp.zeros((RPT, CH), jnp.float32)

    degp = _sc_deg(dst3, ones_deg, zeros_blk)
    xws = _tc_prep(state, W_gcn, degp)
    accp = _sc_scatter(xws, ei[0], ei[1], zeros_blk)
    y = _tc_head(accp, xws, state, degp, b_gcn, W1, b1, W2, b2, W3, b3)
    return y.reshape(N_NODES // 8, 8)


# speedup vs baseline: 1.1247x; 1.0169x over previous
import functools

import jax
import jax.numpy as jnp
from jax import lax
from jax.experimental import pallas as pl
from jax.experimental.pallas import tpu as pltpu
from jax.experimental.pallas import tpu_sc as plsc

N_NODES = 10000
N_EDGES = 320000
CH = 128
HID = 32

NC = 2
NS = 16
NW = NC * NS
EPW = N_EDGES // NW
K = 40
NCHUNK = EPW // K
DEPTH = 6
NPAD = 10240
K_D = 80
NCHUNK_D = EPW // K_D

RPT = NPAD // NS
DEGW = 128

_sc_mesh = plsc.VectorSubcoreMesh(core_axis_name="c", subcore_axis_name="s")


@functools.partial(
    pl.kernel,
    out_type=jax.ShapeDtypeStruct((NC, NPAD, DEGW), jnp.float32),
    mesh=_sc_mesh,
    scratch_types=[
        pltpu.VMEM((NCHUNK_D, K_D), jnp.int32),
        pltpu.VMEM((K_D, DEGW), jnp.float32),
        pltpu.VMEM_SHARED((NPAD, DEGW), jnp.float32),
    ],
)
def _sc_deg(dst_hbm, ones_hbm, zeros_hbm, out_hbm, idx_v, ones_v, deg_sh):
    c = lax.axis_index("c")
    s = lax.axis_index("s")
    wid = c * NS + s
    pltpu.sync_copy(dst_hbm.at[wid], idx_v)
    pltpu.sync_copy(zeros_hbm, deg_sh.at[pl.ds(s * RPT, RPT)])
    pltpu.sync_copy(ones_hbm, ones_v)
    plsc.subcore_barrier()

    def body(j, carry):
        pltpu.sync_copy(ones_v, deg_sh.at[idx_v.at[j]], add=True)
        return carry

    lax.fori_loop(0, NCHUNK_D, body, 0)
    plsc.subcore_barrier()
    pltpu.sync_copy(deg_sh.at[pl.ds(s * RPT, RPT)],
                    out_hbm.at[c, pl.ds(s * RPT, RPT)])


@functools.partial(
    pl.kernel,
    out_type=jax.ShapeDtypeStruct((NC, NPAD, CH), jnp.float32),
    mesh=_sc_mesh,
    scratch_types=[
        pltpu.VMEM((EPW,), jnp.int32),
        pltpu.VMEM((DEPTH, K), jnp.int32),
        pltpu.VMEM((DEPTH, K, CH), jnp.float32),
        pltpu.VMEM_SHARED((NPAD, CH), jnp.float32),
    ] + [pltpu.SemaphoreType.DMA] * (2 * DEPTH),
)
def _sc_scatter(xws_hbm, src_hbm, dst_hbm, zeros_hbm, out_hbm,
                idxs_v, idxd_v, rows_v, acc_sh, *sems_all):
    c = lax.axis_index("c")
    s = lax.axis_index("s")
    wid = c * NS + s
    base0 = wid * EPW
    gsems = sems_all[:DEPTH]
    dsems = sems_all[DEPTH:]
    pltpu.sync_copy(src_hbm.at[pl.ds(base0, EPW)], idxs_v)
    pltpu.sync_copy(zeros_hbm, acc_sh.at[pl.ds(s * RPT, RPT)])
    plsc.subcore_barrier()

    def issue(j, b):
        pltpu.async_copy(dst_hbm.at[pl.ds(base0 + j * K, K)], idxd_v.at[b],
                         dsems[b])
        pltpu.async_copy(xws_hbm.at[idxs_v.at[pl.ds(j * K, K)]],
                         rows_v.at[b], gsems[b])

    def drain_scatter(j, b):
        pltpu.make_async_copy(dst_hbm.at[pl.ds(base0 + j * K, K)],
                              idxd_v.at[b], dsems[b]).wait()
        pltpu.make_async_copy(xws_hbm.at[idxs_v.at[pl.ds(j * K, K)]],
                              rows_v.at[b], gsems[b]).wait()
        pltpu.sync_copy(rows_v.at[b], acc_sh.at[idxd_v.at[b]], add=True)

    for b in range(DEPTH):
        issue(b, b)

    NSTEADY = (NCHUNK - DEPTH + 1) // DEPTH

    def body(i, carry):
        for b in range(DEPTH):
            j = DEPTH * i + b
            drain_scatter(j, b)
            issue(j + DEPTH, b)
        return carry

    lax.fori_loop(0, NSTEADY, body, 0)
    for jj in range(DEPTH * NSTEADY, NCHUNK):
        drain_scatter(jj, jj % DEPTH)
        if jj + DEPTH < NCHUNK:
            issue(jj + DEPTH, jj % DEPTH)
    plsc.subcore_barrier()
    pltpu.sync_copy(acc_sh.at[pl.ds(s * RPT, RPT)],
                    out_hbm.at[c, pl.ds(s * RPT, RPT)])


_R = 2000


def _prep_body(state_ref, w_ref, degp_ref, xws_ref):
    xw = jnp.dot(state_ref[...], w_ref[...], preferred_element_type=jnp.float32)
    degp = degp_ref[...]
    deg = degp[0, :, 0] + degp[1, :, 0] + 1.0
    dinv = lax.rsqrt(deg)
    xws_ref[...] = xw * dinv[:, None]


def _head_body(accp_ref, xws_ref, state_ref, degp_ref, bg_ref,
               w1_ref, b1_ref, w2_ref, b2_ref, w3_ref, b3_ref, out_ref):
    accp = accp_ref[...]
    xws = xws_ref[...]
    degp = degp_ref[...]
    deg = degp[0, :, 0] + degp[1, :, 0] + 1.0
    dinv = lax.rsqrt(deg)[:, None]
    total = dinv * (accp[0] + accp[1] + xws) + bg_ref[...][None, :]
    h = jnp.maximum(total, 0.0) + state_ref[...]
    t = jnp.maximum(jnp.dot(h, w1_ref[...], preferred_element_type=jnp.float32)
                    + b1_ref[...][None, :], 0.0)
    t = jnp.maximum(jnp.dot(t, w2_ref[...], preferred_element_type=jnp.float32)
                    + b2_ref[...][None, :], 0.0)
    z = (jnp.dot(t, w3_ref[...], preferred_element_type=jnp.float32)
         + b3_ref[...][None, :])
    out_ref[...] = (jnp.maximum(z, 0.0) + jnp.log1p(jnp.exp(-jnp.abs(z)))
                    + 1e-20)


_tc_prep = pl.pallas_call(
    _prep_body,
    grid=(N_NODES // _R,),
    in_specs=[
        pl.BlockSpec((_R, CH), lambda i: (i, 0)),
        pl.BlockSpec((CH, CH), lambda i: (0, 0)),
        pl.BlockSpec((NC, _R, DEGW), lambda i: (0, i, 0)),
    ],
    out_specs=pl.BlockSpec((_R, CH), lambda i: (i, 0)),
    out_shape=jax.ShapeDtypeStruct((N_NODES, CH), jnp.float32),
)

_tc_head = pl.pallas_call(
    _head_body,
    grid=(N_NODES // _R,),
    in_specs=[
        pl.BlockSpec((NC, _R, CH), lambda i: (0, i, 0)),
        pl.BlockSpec((_R, CH), lambda i: (i, 0)),
        pl.BlockSpec((_R, CH), lambda i: (i, 0)),
        pl.BlockSpec((NC, _R, DEGW), lambda i: (0, i, 0)),
        pl.BlockSpec((CH,), lambda i: (0,)),
        pl.BlockSpec((CH, HID), lambda i: (0, 0)),
        pl.BlockSpec((HID,), lambda i: (0,)),
        pl.BlockSpec((HID, HID), lambda i: (0, 0)),
        pl.BlockSpec((HID,), lambda i: (0,)),
        pl.BlockSpec((HID, 1), lambda i: (0, 0)),
        pl.BlockSpec((1,), lambda i: (0,)),
    ],
    out_specs=pl.BlockSpec((_R, 1), lambda i: (i, 0)),
    out_shape=jax.ShapeDtypeStruct((N_NODES, 1), jnp.float32),
)


def kernel(state, edge_index, W_gcn, b_gcn, W1, b1, W2, b2, W3, b3):
    ei = edge_index.astype(jnp.int32)
    dst3 = ei[1].reshape(NW, NCHUNK_D, K_D)
    ones_deg = jnp.ones((K_D, DEGW), jnp.float32)
    zeros_blk = jnp.zeros((RPT, CH), jnp.float32)

    degp = _sc_deg(dst3, ones_deg, zeros_blk)
    xws = _tc_prep(state, W_gcn, degp)
    accp = _sc_scatter(xws, ei[0], ei[1], zeros_blk)
    y = _tc_head(accp, xws, state, degp, b_gcn, W1, b1, W2, b2, W3, b3)
    return y.reshape(N_NODES // 8, 8)
